# baseline (speedup 1.0000x reference)
import functools

import jax
import jax.numpy as jnp
from jax import lax
from jax.experimental import pallas as pl
from jax.experimental.pallas import tpu as pltpu
from jax.experimental.pallas import tpu_sc as plsc

NC = 2
NS = 16
NW = NC * NS
CHUNK = 3000


def _round_up(n, m):
    return (n + m - 1) // m * m


@functools.lru_cache(maxsize=None)
def _build(b_pad, dim):
    n_w = b_pad // NW
    mesh = plsc.VectorSubcoreMesh(
        core_axis_name="c", subcore_axis_name="s", num_cores=NC, num_subcores=NS
    )

    @functools.partial(
        pl.kernel,
        mesh=mesh,
        compiler_params=pltpu.CompilerParams(use_tc_tiling_on_sc=False),
        out_type=jax.ShapeDtypeStruct((b_pad, dim), jnp.float32),
        scratch_types=[
            pltpu.VMEM((CHUNK,), jnp.int32),
            pltpu.VMEM((CHUNK, dim), jnp.float32),
            pltpu.SemaphoreType.DMA,
        ],
    )
    def emb_kernel(idx_hbm, tab_hbm, out_hbm, idx_v, rows_v, sem):
        wid = lax.axis_index("s") * NC + lax.axis_index("c")

        def do_chunk(off, size):
            pltpu.sync_copy(idx_hbm.at[pl.ds(off, size)], idx_v.at[pl.ds(0, size)])
            pltpu.async_copy(
                tab_hbm.at[idx_v.at[pl.ds(0, size)]],
                rows_v.at[pl.ds(0, size)],
                sem,
            ).wait()
            pltpu.sync_copy(rows_v.at[pl.ds(0, size)], out_hbm.at[pl.ds(off, size)])

        base = wid * n_w
        k_full = n_w // CHUNK
        rem = n_w % CHUNK
        if k_full:
            @pl.loop(0, k_full)
            def _(i):
                off = pl.multiple_of(base + i * CHUNK, 8)
                do_chunk(off, CHUNK)
        if rem:
            off = pl.multiple_of(base + k_full * CHUNK, 8)
            do_chunk(off, rem)

    return emb_kernel


def _lookup(idx, table):
    b = idx.shape[0]
    dim = table.shape[1]
    b_pad = _round_up(b, NW * 8)
    idx_i = jnp.pad(idx.astype(jnp.int32), (0, b_pad - b))
    out = _build(b_pad, dim)(idx_i, table)
    return out[:b]


def kernel(x, edge_attr, node_table, edge_table):
    return (_lookup(x, node_table), _lookup(edge_attr, edge_table))
